# baseline (speedup 1.0000x reference)
import jax
import jax.numpy as jnp
from jax.experimental import pallas as pl
from jax.experimental.pallas import tpu as pltpu


def _fused_body(comp_ref, adj_ref, seq_ref, w_ref, out_ref):
    g = jnp.dot(adj_ref[...], seq_ref[...],
                preferred_element_type=jnp.float32)
    w = w_ref[...] * comp_ref[0, 0]
    acc = jnp.dot(g, w, preferred_element_type=jnp.float32)
    out_ref[...] = jnp.maximum(acc, 0.0)


def kernel(seqs, adjs, comp, weight):
    seq = seqs[0]
    adj = adjs[0]
    w = weight[0]
    n, in_ft = seq.shape
    out_ft = w.shape[1]

    bi = 200
    grid = (n // bi,)
    out = pl.pallas_call(
        _fused_body,
        grid=grid,
        in_specs=[
            pl.BlockSpec(memory_space=pltpu.SMEM),
            pl.BlockSpec((bi, n), lambda i: (i, 0)),
            pl.BlockSpec((n, in_ft), lambda i: (0, 0)),
            pl.BlockSpec((in_ft, out_ft), lambda i: (0, 0)),
        ],
        out_specs=pl.BlockSpec((bi, out_ft), lambda i: (i, 0)),
        out_shape=jax.ShapeDtypeStruct((n, out_ft), jnp.float32),
        compiler_params=pltpu.CompilerParams(
            dimension_semantics=("arbitrary",)),
    )(comp, adj, seq, w)
    return out


# speedup vs baseline: 1.0169x; 1.0169x over previous
import jax
import jax.numpy as jnp
from jax.experimental import pallas as pl
from jax.experimental.pallas import tpu as pltpu


def _fused_body(comp_ref, adj_ref, seq_ref, w_ref, out_ref):
    g = jnp.dot(adj_ref[...], seq_ref[...],
                preferred_element_type=jnp.float32)
    w = w_ref[...] * comp_ref[0, 0]
    acc = jnp.dot(g, w, preferred_element_type=jnp.float32)
    out_ref[...] = jnp.maximum(acc, 0.0)


def kernel(seqs, adjs, comp, weight):
    seq = seqs[0]
    adj = adjs[0]
    w = weight[0]
    n, in_ft = seq.shape
    out_ft = w.shape[1]

    bi = 400
    grid = (n // bi,)
    out = pl.pallas_call(
        _fused_body,
        grid=grid,
        in_specs=[
            pl.BlockSpec(memory_space=pltpu.SMEM),
            pl.BlockSpec((bi, n), lambda i: (i, 0)),
            pl.BlockSpec((n, in_ft), lambda i: (0, 0)),
            pl.BlockSpec((in_ft, out_ft), lambda i: (0, 0)),
        ],
        out_specs=pl.BlockSpec((bi, out_ft), lambda i: (i, 0)),
        out_shape=jax.ShapeDtypeStruct((n, out_ft), jnp.float32),
        compiler_params=pltpu.CompilerParams(
            dimension_semantics=("arbitrary",)),
    )(comp, adj, seq, w)
    return out
